# baseline (speedup 1.0000x reference)
import functools

import jax
import jax.numpy as jnp
from jax import lax
from jax.experimental import pallas as pl
from jax.experimental.pallas import tpu as pltpu
from jax.experimental.pallas import tpu_sc as plsc

NUM_EXPERTS = 8
INPUT_DIM = 768
HIDDEN_DIM = 768
NUM_TOKENS = 32768
HALF_K = INPUT_DIM // 2

RB = 1024
NB = NUM_TOKENS // RB
BT = 512
NBLK = NUM_TOKENS // BT + NUM_EXPERTS
PAD_N = NBLK * BT


def _wcomb_kernel(ws_ref, we_ref, bs_ref, be_ref, wc_ref, bc_ref):
    comb = ws_ref[...] + we_ref[0]
    wc_ref[0, 0] = comb[:HALF_K, :].astype(jnp.bfloat16)
    wc_ref[0, 1] = comb[HALF_K:, :].astype(jnp.bfloat16)
    bc_ref[0] = bs_ref[...] + be_ref[0]


def _combine_weights(W_shared, b_shared, W_experts, b_experts):
    return pl.pallas_call(
        _wcomb_kernel,
        grid=(NUM_EXPERTS,),
        in_specs=[
            pl.BlockSpec((INPUT_DIM, HIDDEN_DIM), lambda i: (0, 0)),
            pl.BlockSpec((1, INPUT_DIM, HIDDEN_DIM), lambda i: (i, 0, 0)),
            pl.BlockSpec((1, HIDDEN_DIM), lambda i: (0, 0)),
            pl.BlockSpec((1, 1, HIDDEN_DIM), lambda i: (i, 0, 0)),
        ],
        out_specs=[
            pl.BlockSpec((1, 2, HALF_K, HIDDEN_DIM), lambda i: (i, 0, 0, 0)),
            pl.BlockSpec((1, 1, HIDDEN_DIM), lambda i: (i, 0, 0)),
        ],
        out_shape=[
            jax.ShapeDtypeStruct((NUM_EXPERTS, 2, HALF_K, HIDDEN_DIM),
                                 jnp.bfloat16),
            jax.ShapeDtypeStruct((NUM_EXPERTS, 1, HIDDEN_DIM), jnp.float32),
        ],
    )(W_shared, W_experts, b_shared.reshape(1, -1),
      b_experts.reshape(NUM_EXPERTS, 1, HIDDEN_DIM))


def _router_kernel(x_ref, wr_ref, br_ref, tril_ref, e_ref, cnt_ref, loc_ref,
                   xb_ref):
    x = x_ref[...]
    u_lo = lax.bitcast_convert_type(x[:, :HALF_K], jnp.uint32)
    u_hi = lax.bitcast_convert_type(x[:, HALF_K:], jnp.uint32)
    rnd = lambda u: u + jnp.uint32(0x8000)
    packed = (rnd(u_hi) & jnp.uint32(0xFFFF0000)) | (rnd(u_lo) >> 16)
    xb_ref[...] = lax.bitcast_convert_type(packed, jnp.int32)
    logits = jnp.dot(x, wr_ref[...], preferred_element_type=jnp.float32)
    logits = logits + br_ref[...]
    e = jnp.argmax(logits, axis=-1).astype(jnp.int32)
    e2 = e[:, None]
    e_ref[0] = e2
    oh = (e2 == lax.broadcasted_iota(jnp.int32, (RB, NUM_EXPERTS), 1))
    ohf = oh.astype(jnp.float32)
    cnt_ref[0] = ohf.sum(axis=0, keepdims=True).astype(jnp.int32)
    excl = jnp.dot(tril_ref[...], ohf,
                   preferred_element_type=jnp.float32)
    loc = (excl * ohf).sum(axis=1)
    loc_ref[0] = loc.astype(jnp.int32)[:, None]


def _route(x, W_router, b_router):
    r = lax.broadcasted_iota(jnp.int32, (RB, RB), 0)
    c = lax.broadcasted_iota(jnp.int32, (RB, RB), 1)
    tril = (c < r).astype(jnp.float32)
    return pl.pallas_call(
        _router_kernel,
        grid=(NB,),
        in_specs=[
            pl.BlockSpec((RB, INPUT_DIM), lambda j: (j, 0)),
            pl.BlockSpec((INPUT_DIM, NUM_EXPERTS), lambda j: (0, 0)),
            pl.BlockSpec((1, NUM_EXPERTS), lambda j: (0, 0)),
            pl.BlockSpec((RB, RB), lambda j: (0, 0)),
        ],
        out_specs=[
            pl.BlockSpec((1, RB, 1), lambda j: (j, 0, 0)),
            pl.BlockSpec((1, 1, NUM_EXPERTS), lambda j: (j, 0, 0)),
            pl.BlockSpec((1, RB, 1), lambda j: (j, 0, 0)),
            pl.BlockSpec((RB, HALF_K), lambda j: (j, 0)),
        ],
        out_shape=[
            jax.ShapeDtypeStruct((NB, RB, 1), jnp.int32),
            jax.ShapeDtypeStruct((NB, 1, NUM_EXPERTS), jnp.int32),
            jax.ShapeDtypeStruct((NB, RB, 1), jnp.int32),
            jax.ShapeDtypeStruct((NUM_TOKENS, HALF_K), jnp.int32),
        ],
    )(x, W_router, b_router.reshape(1, -1), tril)


PB = 4
PR = NB // PB


def _pos_kernel(e_ref, cnt_ref, loc_ref, pos_ref, g_ref):
    j = pl.program_id(0)
    c = cnt_ref[:, 0, :]
    totals = c.sum(axis=0)
    padded = ((totals + BT - 1) // BT) * BT
    em = lax.broadcasted_iota(jnp.int32, (NUM_EXPERTS, NUM_EXPERTS), 1)
    en = lax.broadcasted_iota(jnp.int32, (NUM_EXPERTS, NUM_EXPERTS), 0)
    start = jnp.where(em < en, padded[None, :], 0).sum(axis=1)

    biota = lax.broadcasted_iota(jnp.int32, (NB, NUM_EXPERTS), 0)
    bases = []
    for k in range(PR):
        blk = j * PR + k
        rank_base = jnp.where(biota < blk, c, 0).sum(axis=0)
        bases.append(jnp.broadcast_to((start + rank_base)[None, :],
                                      (RB, NUM_EXPERTS)))
    base = jnp.concatenate(bases, axis=0)

    e2 = e_ref[0]
    oh = (e2 == lax.broadcasted_iota(jnp.int32,
                                     (PR * RB, NUM_EXPERTS), 1))
    pos = jnp.where(oh, base, 0).sum(axis=1) + loc_ref[0, :, 0]
    pos_ref[0] = pos[:, None]

    bvec = lax.broadcasted_iota(jnp.int32, (NBLK, NUM_EXPERTS), 0) * BT
    g = (start[None, :] <= bvec).astype(jnp.int32).sum(axis=1) - 1
    g_ref[...] = g[None, :]


def _positions(e, counts, local):
    e4 = e.reshape(PB, PR * RB, 1)
    loc4 = local.reshape(PB, PR * RB, 1)
    return pl.pallas_call(
        _pos_kernel,
        grid=(PB,),
        in_specs=[
            pl.BlockSpec((1, PR * RB, 1), lambda j: (j, 0, 0)),
            pl.BlockSpec((NB, 1, NUM_EXPERTS), lambda j: (0, 0, 0)),
            pl.BlockSpec((1, PR * RB, 1), lambda j: (j, 0, 0)),
        ],
        out_specs=[
            pl.BlockSpec((1, PR * RB, 1), lambda j: (j, 0, 0)),
            pl.BlockSpec((1, NBLK), lambda j: (0, 0)),
        ],
        out_shape=[
            jax.ShapeDtypeStruct((PB, PR * RB, 1), jnp.int32),
            jax.ShapeDtypeStruct((1, NBLK), jnp.int32),
        ],
    )(e4, counts, loc4)


def _sc_worker_id():
    info = plsc.get_sparse_core_info()
    return (lax.axis_index("s") * info.num_cores + lax.axis_index("c"),
            info.num_cores * info.num_subcores)


_SCAT_CHUNK = 128
_GATH_CHUNK = 64


def _scatter_rows(x, pos, out_rows):
    n, dim = x.shape
    nchunk_total = n // _SCAT_CHUNK
    pos2d = pos.reshape(nchunk_total, _SCAT_CHUNK)
    mesh = plsc.VectorSubcoreMesh(core_axis_name="c", subcore_axis_name="s")

    @functools.partial(
        pl.kernel, mesh=mesh,
        out_type=jax.ShapeDtypeStruct((out_rows, dim), x.dtype),
        scratch_types=[
            pltpu.VMEM((nchunk_total // 32, _SCAT_CHUNK), jnp.int32),
            pltpu.VMEM((_SCAT_CHUNK, dim), x.dtype),
            pltpu.VMEM((_SCAT_CHUNK, dim), x.dtype),
            pltpu.SemaphoreType.DMA,
        ],
    )
    def k(x_hbm, pos_hbm, out_hbm, idx_v, rows0, rows1, sem):
        wid, nw = _sc_worker_id()
        nchunks = nchunk_total // nw
        crow = wid * nchunks
        pltpu.sync_copy(pos_hbm.at[pl.ds(crow, nchunks)], idx_v)
        bufs = (rows0, rows1)
        pltpu.sync_copy(x_hbm.at[pl.ds(crow * _SCAT_CHUNK, _SCAT_CHUNK)],
                        rows0)
        for it in range(nchunks):
            h = pltpu.async_copy(bufs[it % 2], out_hbm.at[idx_v.at[it]], sem)
            if it + 1 < nchunks:
                nxt = (crow + it + 1) * _SCAT_CHUNK
                pltpu.sync_copy(x_hbm.at[pl.ds(nxt, _SCAT_CHUNK)],
                                bufs[(it + 1) % 2])
            h.wait()

    return k(x, pos2d)


def _gather_rows(ys, pos, n):
    dim = ys.shape[1]
    nchunk_total = n // _GATH_CHUNK
    pos2d = pos.reshape(nchunk_total, _GATH_CHUNK)
    mesh = plsc.VectorSubcoreMesh(core_axis_name="c", subcore_axis_name="s")

    @functools.partial(
        pl.kernel, mesh=mesh,
        out_type=jax.ShapeDtypeStruct((n, dim), ys.dtype),
        scratch_types=[
            pltpu.VMEM((nchunk_total // 32, _GATH_CHUNK), jnp.int32),
            pltpu.VMEM((_GATH_CHUNK, dim), ys.dtype),
            pltpu.VMEM((_GATH_CHUNK, dim), ys.dtype),
            pltpu.SemaphoreType.DMA,
        ],
    )
    def k(ys_hbm, pos_hbm, out_hbm, idx_v, rows0, rows1, sem):
        wid, nw = _sc_worker_id()
        nchunks = nchunk_total // nw
        crow = wid * nchunks
        pltpu.sync_copy(pos_hbm.at[pl.ds(crow, nchunks)], idx_v)
        bufs = (rows0, rows1)
        h = pltpu.async_copy(ys_hbm.at[idx_v.at[0]], rows0, sem)
        for it in range(nchunks):
            h.wait()
            if it + 1 < nchunks:
                h = pltpu.async_copy(ys_hbm.at[idx_v.at[it + 1]],
                                     bufs[(it + 1) % 2], sem)
            out_base = (crow + it) * _GATH_CHUNK
            pltpu.sync_copy(bufs[it % 2],
                            out_hbm.at[pl.ds(out_base, _GATH_CHUNK)])

    return k(ys, pos2d)


def _mm_kernel(g_ref, xs_ref, wc_ref, bc_ref, ys_ref):
    v = lax.bitcast_convert_type(xs_ref[...], jnp.uint32)
    xlo = lax.bitcast_convert_type(v << 16, jnp.float32).astype(jnp.bfloat16)
    xhi = lax.bitcast_convert_type(v & jnp.uint32(0xFFFF0000),
                                   jnp.float32).astype(jnp.bfloat16)
    acc = jnp.dot(xlo, wc_ref[0, 0], preferred_element_type=jnp.float32)
    acc += jnp.dot(xhi, wc_ref[0, 1], preferred_element_type=jnp.float32)
    ys_ref[...] = acc + bc_ref[0]


def _grouped_matmul(xs, g, W_comb, b_comb):
    grid_spec = pltpu.PrefetchScalarGridSpec(
        num_scalar_prefetch=1,
        grid=(NBLK,),
        in_specs=[
            pl.BlockSpec((BT, HALF_K), lambda b, g: (b, 0)),
            pl.BlockSpec((1, 2, HALF_K, HIDDEN_DIM),
                         lambda b, g: (g[b], 0, 0, 0)),
            pl.BlockSpec((1, 1, HIDDEN_DIM), lambda b, g: (g[b], 0, 0)),
        ],
        out_specs=pl.BlockSpec((BT, HIDDEN_DIM), lambda b, g: (b, 0)),
    )
    return pl.pallas_call(
        _mm_kernel,
        grid_spec=grid_spec,
        out_shape=jax.ShapeDtypeStruct((PAD_N, HIDDEN_DIM), jnp.float32),
    )(g, xs, W_comb, b_comb)


def kernel(x, W_shared, b_shared, W_experts, b_experts, W_router, b_router):
    W_comb, b_comb = _combine_weights(W_shared, b_shared, W_experts,
                                      b_experts)
    e, counts, local, xb = _route(x, W_router, b_router)
    pos3, g2 = _positions(e, counts, local)
    pos = pos3.reshape(NUM_TOKENS)
    g = g2.reshape(NBLK)
    xs = _scatter_rows(xb, pos, PAD_N)
    ys = _grouped_matmul(xs, g, W_comb, b_comb)
    out = _gather_rows(ys, pos, NUM_TOKENS)
    return out


# speedup vs baseline: 1.0399x; 1.0399x over previous
import functools

import jax
import jax.numpy as jnp
from jax import lax
from jax.experimental import pallas as pl
from jax.experimental.pallas import tpu as pltpu
from jax.experimental.pallas import tpu_sc as plsc

NUM_EXPERTS = 8
INPUT_DIM = 768
HIDDEN_DIM = 768
NUM_TOKENS = 32768
HALF_K = INPUT_DIM // 2

RB = 1024
NB = NUM_TOKENS // RB
BT = 1024
NBLK = NUM_TOKENS // BT + NUM_EXPERTS
PAD_N = NBLK * BT


def _wcomb_kernel(ws_ref, we_ref, bs_ref, be_ref, wc_ref, bc_ref):
    comb = ws_ref[...] + we_ref[0]
    wc_ref[0, 0] = comb[:HALF_K, :].astype(jnp.bfloat16)
    wc_ref[0, 1] = comb[HALF_K:, :].astype(jnp.bfloat16)
    bc_ref[0] = bs_ref[...] + be_ref[0]


def _combine_weights(W_shared, b_shared, W_experts, b_experts):
    return pl.pallas_call(
        _wcomb_kernel,
        grid=(NUM_EXPERTS,),
        in_specs=[
            pl.BlockSpec((INPUT_DIM, HIDDEN_DIM), lambda i: (0, 0)),
            pl.BlockSpec((1, INPUT_DIM, HIDDEN_DIM), lambda i: (i, 0, 0)),
            pl.BlockSpec((1, HIDDEN_DIM), lambda i: (0, 0)),
            pl.BlockSpec((1, 1, HIDDEN_DIM), lambda i: (i, 0, 0)),
        ],
        out_specs=[
            pl.BlockSpec((1, 2, HALF_K, HIDDEN_DIM), lambda i: (i, 0, 0, 0)),
            pl.BlockSpec((1, 1, HIDDEN_DIM), lambda i: (i, 0, 0)),
        ],
        out_shape=[
            jax.ShapeDtypeStruct((NUM_EXPERTS, 2, HALF_K, HIDDEN_DIM),
                                 jnp.bfloat16),
            jax.ShapeDtypeStruct((NUM_EXPERTS, 1, HIDDEN_DIM), jnp.float32),
        ],
    )(W_shared, W_experts, b_shared.reshape(1, -1),
      b_experts.reshape(NUM_EXPERTS, 1, HIDDEN_DIM))


def _router_kernel(x_ref, wr_ref, br_ref, tril_ref, e_ref, cnt_ref, loc_ref,
                   xb_ref):
    x = x_ref[...]
    u_lo = lax.bitcast_convert_type(x[:, :HALF_K], jnp.uint32)
    u_hi = lax.bitcast_convert_type(x[:, HALF_K:], jnp.uint32)
    rnd = lambda u: u + jnp.uint32(0x8000)
    packed = (rnd(u_hi) & jnp.uint32(0xFFFF0000)) | (rnd(u_lo) >> 16)
    xb_ref[...] = lax.bitcast_convert_type(packed, jnp.int32)
    logits = jnp.dot(x, wr_ref[...], preferred_element_type=jnp.float32)
    logits = logits + br_ref[...]
    e = jnp.argmax(logits, axis=-1).astype(jnp.int32)
    e2 = e[:, None]
    e_ref[0] = e2
    oh = (e2 == lax.broadcasted_iota(jnp.int32, (RB, NUM_EXPERTS), 1))
    ohf = oh.astype(jnp.float32)
    cnt_ref[0] = ohf.sum(axis=0, keepdims=True).astype(jnp.int32)
    excl = jnp.dot(tril_ref[...], ohf,
                   preferred_element_type=jnp.float32)
    loc = (excl * ohf).sum(axis=1)
    loc_ref[0] = loc.astype(jnp.int32)[:, None]


def _route(x, W_router, b_router):
    r = lax.broadcasted_iota(jnp.int32, (RB, RB), 0)
    c = lax.broadcasted_iota(jnp.int32, (RB, RB), 1)
    tril = (c < r).astype(jnp.float32)
    return pl.pallas_call(
        _router_kernel,
        grid=(NB,),
        in_specs=[
            pl.BlockSpec((RB, INPUT_DIM), lambda j: (j, 0)),
            pl.BlockSpec((INPUT_DIM, NUM_EXPERTS), lambda j: (0, 0)),
            pl.BlockSpec((1, NUM_EXPERTS), lambda j: (0, 0)),
            pl.BlockSpec((RB, RB), lambda j: (0, 0)),
        ],
        out_specs=[
            pl.BlockSpec((1, RB, 1), lambda j: (j, 0, 0)),
            pl.BlockSpec((1, 1, NUM_EXPERTS), lambda j: (j, 0, 0)),
            pl.BlockSpec((1, RB, 1), lambda j: (j, 0, 0)),
            pl.BlockSpec((RB, HALF_K), lambda j: (j, 0)),
        ],
        out_shape=[
            jax.ShapeDtypeStruct((NB, RB, 1), jnp.int32),
            jax.ShapeDtypeStruct((NB, 1, NUM_EXPERTS), jnp.int32),
            jax.ShapeDtypeStruct((NB, RB, 1), jnp.int32),
            jax.ShapeDtypeStruct((NUM_TOKENS, HALF_K), jnp.int32),
        ],
    )(x, W_router, b_router.reshape(1, -1), tril)


PB = 4
PR = NB // PB


def _pos_kernel(e_ref, cnt_ref, loc_ref, pos_ref, g_ref):
    j = pl.program_id(0)
    c = cnt_ref[:, 0, :]
    totals = c.sum(axis=0)
    padded = ((totals + BT - 1) // BT) * BT
    em = lax.broadcasted_iota(jnp.int32, (NUM_EXPERTS, NUM_EXPERTS), 1)
    en = lax.broadcasted_iota(jnp.int32, (NUM_EXPERTS, NUM_EXPERTS), 0)
    start = jnp.where(em < en, padded[None, :], 0).sum(axis=1)

    biota = lax.broadcasted_iota(jnp.int32, (NB, NUM_EXPERTS), 0)
    bases = []
    for k in range(PR):
        blk = j * PR + k
        rank_base = jnp.where(biota < blk, c, 0).sum(axis=0)
        bases.append(jnp.broadcast_to((start + rank_base)[None, :],
                                      (RB, NUM_EXPERTS)))
    base = jnp.concatenate(bases, axis=0)

    e2 = e_ref[0]
    oh = (e2 == lax.broadcasted_iota(jnp.int32,
                                     (PR * RB, NUM_EXPERTS), 1))
    pos = jnp.where(oh, base, 0).sum(axis=1) + loc_ref[0, :, 0]
    pos_ref[0] = pos[:, None]

    bvec = lax.broadcasted_iota(jnp.int32, (NBLK, NUM_EXPERTS), 0) * BT
    g = (start[None, :] <= bvec).astype(jnp.int32).sum(axis=1) - 1
    g_ref[...] = g[None, :]


def _positions(e, counts, local):
    e4 = e.reshape(PB, PR * RB, 1)
    loc4 = local.reshape(PB, PR * RB, 1)
    return pl.pallas_call(
        _pos_kernel,
        grid=(PB,),
        in_specs=[
            pl.BlockSpec((1, PR * RB, 1), lambda j: (j, 0, 0)),
            pl.BlockSpec((NB, 1, NUM_EXPERTS), lambda j: (0, 0, 0)),
            pl.BlockSpec((1, PR * RB, 1), lambda j: (j, 0, 0)),
        ],
        out_specs=[
            pl.BlockSpec((1, PR * RB, 1), lambda j: (j, 0, 0)),
            pl.BlockSpec((1, NBLK), lambda j: (0, 0)),
        ],
        out_shape=[
            jax.ShapeDtypeStruct((PB, PR * RB, 1), jnp.int32),
            jax.ShapeDtypeStruct((1, NBLK), jnp.int32),
        ],
    )(e4, counts, loc4)


def _sc_worker_id():
    info = plsc.get_sparse_core_info()
    return (lax.axis_index("s") * info.num_cores + lax.axis_index("c"),
            info.num_cores * info.num_subcores)


_SCAT_CHUNK = 128
_GATH_CHUNK = 64


def _scatter_rows(x, pos, out_rows):
    n, dim = x.shape
    nchunk_total = n // _SCAT_CHUNK
    pos2d = pos.reshape(nchunk_total, _SCAT_CHUNK)
    mesh = plsc.VectorSubcoreMesh(core_axis_name="c", subcore_axis_name="s")

    @functools.partial(
        pl.kernel, mesh=mesh,
        out_type=jax.ShapeDtypeStruct((out_rows, dim), x.dtype),
        scratch_types=[
            pltpu.VMEM((nchunk_total // 32, _SCAT_CHUNK), jnp.int32),
            pltpu.VMEM((_SCAT_CHUNK, dim), x.dtype),
            pltpu.VMEM((_SCAT_CHUNK, dim), x.dtype),
            pltpu.SemaphoreType.DMA,
        ],
    )
    def k(x_hbm, pos_hbm, out_hbm, idx_v, rows0, rows1, sem):
        wid, nw = _sc_worker_id()
        nchunks = nchunk_total // nw
        crow = wid * nchunks
        pltpu.sync_copy(pos_hbm.at[pl.ds(crow, nchunks)], idx_v)
        bufs = (rows0, rows1)
        pltpu.sync_copy(x_hbm.at[pl.ds(crow * _SCAT_CHUNK, _SCAT_CHUNK)],
                        rows0)
        for it in range(nchunks):
            h = pltpu.async_copy(bufs[it % 2], out_hbm.at[idx_v.at[it]], sem)
            if it + 1 < nchunks:
                nxt = (crow + it + 1) * _SCAT_CHUNK
                pltpu.sync_copy(x_hbm.at[pl.ds(nxt, _SCAT_CHUNK)],
                                bufs[(it + 1) % 2])
            h.wait()

    return k(x, pos2d)


def _gather_rows(ys, pos, n):
    dim = ys.shape[1]
    nchunk_total = n // _GATH_CHUNK
    pos2d = pos.reshape(nchunk_total, _GATH_CHUNK)
    mesh = plsc.VectorSubcoreMesh(core_axis_name="c", subcore_axis_name="s")

    @functools.partial(
        pl.kernel, mesh=mesh,
        out_type=jax.ShapeDtypeStruct((n, dim), ys.dtype),
        scratch_types=[
            pltpu.VMEM((nchunk_total // 32, _GATH_CHUNK), jnp.int32),
            pltpu.VMEM((_GATH_CHUNK, dim), ys.dtype),
            pltpu.VMEM((_GATH_CHUNK, dim), ys.dtype),
            pltpu.SemaphoreType.DMA,
        ],
    )
    def k(ys_hbm, pos_hbm, out_hbm, idx_v, rows0, rows1, sem):
        wid, nw = _sc_worker_id()
        nchunks = nchunk_total // nw
        crow = wid * nchunks
        pltpu.sync_copy(pos_hbm.at[pl.ds(crow, nchunks)], idx_v)
        bufs = (rows0, rows1)
        h = pltpu.async_copy(ys_hbm.at[idx_v.at[0]], rows0, sem)
        for it in range(nchunks):
            h.wait()
            if it + 1 < nchunks:
                h = pltpu.async_copy(ys_hbm.at[idx_v.at[it + 1]],
                                     bufs[(it + 1) % 2], sem)
            out_base = (crow + it) * _GATH_CHUNK
            pltpu.sync_copy(bufs[it % 2],
                            out_hbm.at[pl.ds(out_base, _GATH_CHUNK)])

    return k(ys, pos2d)


def _mm_kernel(g_ref, xs_ref, wc_ref, bc_ref, ys_ref):
    v = lax.bitcast_convert_type(xs_ref[...], jnp.uint32)
    xlo = lax.bitcast_convert_type(v << 16, jnp.float32).astype(jnp.bfloat16)
    xhi = lax.bitcast_convert_type(v & jnp.uint32(0xFFFF0000),
                                   jnp.float32).astype(jnp.bfloat16)
    acc = jnp.dot(xlo, wc_ref[0, 0], preferred_element_type=jnp.float32)
    acc += jnp.dot(xhi, wc_ref[0, 1], preferred_element_type=jnp.float32)
    ys_ref[...] = acc + bc_ref[0]


def _grouped_matmul(xs, g, W_comb, b_comb):
    grid_spec = pltpu.PrefetchScalarGridSpec(
        num_scalar_prefetch=1,
        grid=(NBLK,),
        in_specs=[
            pl.BlockSpec((BT, HALF_K), lambda b, g: (b, 0)),
            pl.BlockSpec((1, 2, HALF_K, HIDDEN_DIM),
                         lambda b, g: (g[b], 0, 0, 0)),
            pl.BlockSpec((1, 1, HIDDEN_DIM), lambda b, g: (g[b], 0, 0)),
        ],
        out_specs=pl.BlockSpec((BT, HIDDEN_DIM), lambda b, g: (b, 0)),
    )
    return pl.pallas_call(
        _mm_kernel,
        grid_spec=grid_spec,
        out_shape=jax.ShapeDtypeStruct((PAD_N, HIDDEN_DIM), jnp.float32),
    )(g, xs, W_comb, b_comb)


def kernel(x, W_shared, b_shared, W_experts, b_experts, W_router, b_router):
    W_comb, b_comb = _combine_weights(W_shared, b_shared, W_experts,
                                      b_experts)
    e, counts, local, xb = _route(x, W_router, b_router)
    pos3, g2 = _positions(e, counts, local)
    pos = pos3.reshape(NUM_TOKENS)
    g = g2.reshape(NBLK)
    xs = _scatter_rows(xb, pos, PAD_N)
    ys = _grouped_matmul(xs, g, W_comb, b_comb)
    out = _gather_rows(ys, pos, NUM_TOKENS)
    return out


# speedup vs baseline: 1.0465x; 1.0063x over previous
import functools

import numpy as np
import jax
import jax.numpy as jnp
from jax import lax
from jax.experimental import pallas as pl
from jax.experimental.pallas import tpu as pltpu
from jax.experimental.pallas import tpu_sc as plsc

NUM_EXPERTS = 8
INPUT_DIM = 768
HIDDEN_DIM = 768
NUM_TOKENS = 32768
HALF_K = INPUT_DIM // 2

RB = 1024
NB = NUM_TOKENS // RB
BT = 1024
NBLK = NUM_TOKENS // BT + NUM_EXPERTS
PAD_N = NBLK * BT


def _wcomb_kernel(ws_ref, we_ref, bs_ref, be_ref, wc_ref, bc_ref):
    comb = ws_ref[...] + we_ref[0]
    wc_ref[0, 0] = comb[:HALF_K, :].astype(jnp.bfloat16)
    wc_ref[0, 1] = comb[HALF_K:, :].astype(jnp.bfloat16)
    bc_ref[0] = bs_ref[...] + be_ref[0]


def _combine_weights(W_shared, b_shared, W_experts, b_experts):
    return pl.pallas_call(
        _wcomb_kernel,
        grid=(NUM_EXPERTS,),
        in_specs=[
            pl.BlockSpec((INPUT_DIM, HIDDEN_DIM), lambda i: (0, 0)),
            pl.BlockSpec((1, INPUT_DIM, HIDDEN_DIM), lambda i: (i, 0, 0)),
            pl.BlockSpec((1, HIDDEN_DIM), lambda i: (0, 0)),
            pl.BlockSpec((1, 1, HIDDEN_DIM), lambda i: (i, 0, 0)),
        ],
        out_specs=[
            pl.BlockSpec((1, 2, HALF_K, HIDDEN_DIM), lambda i: (i, 0, 0, 0)),
            pl.BlockSpec((1, 1, HIDDEN_DIM), lambda i: (i, 0, 0)),
        ],
        out_shape=[
            jax.ShapeDtypeStruct((NUM_EXPERTS, 2, HALF_K, HIDDEN_DIM),
                                 jnp.bfloat16),
            jax.ShapeDtypeStruct((NUM_EXPERTS, 1, HIDDEN_DIM), jnp.float32),
        ],
    )(W_shared, W_experts, b_shared.reshape(1, -1),
      b_experts.reshape(NUM_EXPERTS, 1, HIDDEN_DIM))


def _router_kernel(x_ref, wr_ref, br_ref, tril_ref, e_ref, cnt_ref, loc_ref,
                   xb_ref):
    x = x_ref[...]
    u_lo = lax.bitcast_convert_type(x[:, :HALF_K], jnp.uint32)
    u_hi = lax.bitcast_convert_type(x[:, HALF_K:], jnp.uint32)
    rnd = lambda u: u + jnp.uint32(0x8000)
    packed = (rnd(u_hi) & jnp.uint32(0xFFFF0000)) | (rnd(u_lo) >> 16)
    xb_ref[...] = lax.bitcast_convert_type(packed, jnp.int32)
    logits = jnp.dot(x, wr_ref[...], preferred_element_type=jnp.float32)
    logits = logits + br_ref[...]
    e = jnp.argmax(logits, axis=-1).astype(jnp.int32)
    e2 = e[:, None]
    e_ref[0] = e2
    oh = (e2 == lax.broadcasted_iota(jnp.int32, (RB, NUM_EXPERTS), 1))
    ohf = oh.astype(jnp.float32)
    cnt_ref[0] = ohf.sum(axis=0, keepdims=True).astype(jnp.int32)
    excl = jnp.dot(tril_ref[...], ohf,
                   preferred_element_type=jnp.float32)
    loc = (excl * ohf).sum(axis=1)
    loc_ref[0] = loc.astype(jnp.int32)[:, None]


def _route(x, W_router, b_router):
    tril = jnp.asarray(np.tril(np.ones((RB, RB), np.float32), -1))
    return pl.pallas_call(
        _router_kernel,
        grid=(NB,),
        in_specs=[
            pl.BlockSpec((RB, INPUT_DIM), lambda j: (j, 0)),
            pl.BlockSpec((INPUT_DIM, NUM_EXPERTS), lambda j: (0, 0)),
            pl.BlockSpec((1, NUM_EXPERTS), lambda j: (0, 0)),
            pl.BlockSpec((RB, RB), lambda j: (0, 0)),
        ],
        out_specs=[
            pl.BlockSpec((1, RB, 1), lambda j: (j, 0, 0)),
            pl.BlockSpec((1, 1, NUM_EXPERTS), lambda j: (j, 0, 0)),
            pl.BlockSpec((1, RB, 1), lambda j: (j, 0, 0)),
            pl.BlockSpec((RB, HALF_K), lambda j: (j, 0)),
        ],
        out_shape=[
            jax.ShapeDtypeStruct((NB, RB, 1), jnp.int32),
            jax.ShapeDtypeStruct((NB, 1, NUM_EXPERTS), jnp.int32),
            jax.ShapeDtypeStruct((NB, RB, 1), jnp.int32),
            jax.ShapeDtypeStruct((NUM_TOKENS, HALF_K), jnp.int32),
        ],
    )(x, W_router, b_router.reshape(1, -1), tril)


PB = 4
PR = NB // PB


def _pos_kernel(e_ref, cnt_ref, loc_ref, pos_ref, g_ref):
    j = pl.program_id(0)
    c = cnt_ref[:, 0, :]
    totals = c.sum(axis=0)
    padded = ((totals + BT - 1) // BT) * BT
    em = lax.broadcasted_iota(jnp.int32, (NUM_EXPERTS, NUM_EXPERTS), 1)
    en = lax.broadcasted_iota(jnp.int32, (NUM_EXPERTS, NUM_EXPERTS), 0)
    start = jnp.where(em < en, padded[None, :], 0).sum(axis=1)

    biota = lax.broadcasted_iota(jnp.int32, (NB, NUM_EXPERTS), 0)
    bases = []
    for k in range(PR):
        blk = j * PR + k
        rank_base = jnp.where(biota < blk, c, 0).sum(axis=0)
        bases.append(jnp.broadcast_to((start + rank_base)[None, :],
                                      (RB, NUM_EXPERTS)))
    base = jnp.concatenate(bases, axis=0)

    e2 = e_ref[0]
    oh = (e2 == lax.broadcasted_iota(jnp.int32,
                                     (PR * RB, NUM_EXPERTS), 1))
    pos = jnp.where(oh, base, 0).sum(axis=1) + loc_ref[0, :, 0]
    pos_ref[0] = pos[:, None]

    bvec = lax.broadcasted_iota(jnp.int32, (NBLK, NUM_EXPERTS), 0) * BT
    g = (start[None, :] <= bvec).astype(jnp.int32).sum(axis=1) - 1
    g_ref[...] = g[None, :]


def _positions(e, counts, local):
    e4 = e.reshape(PB, PR * RB, 1)
    loc4 = local.reshape(PB, PR * RB, 1)
    return pl.pallas_call(
        _pos_kernel,
        grid=(PB,),
        in_specs=[
            pl.BlockSpec((1, PR * RB, 1), lambda j: (j, 0, 0)),
            pl.BlockSpec((NB, 1, NUM_EXPERTS), lambda j: (0, 0, 0)),
            pl.BlockSpec((1, PR * RB, 1), lambda j: (j, 0, 0)),
        ],
        out_specs=[
            pl.BlockSpec((1, PR * RB, 1), lambda j: (j, 0, 0)),
            pl.BlockSpec((1, NBLK), lambda j: (0, 0)),
        ],
        out_shape=[
            jax.ShapeDtypeStruct((PB, PR * RB, 1), jnp.int32),
            jax.ShapeDtypeStruct((1, NBLK), jnp.int32),
        ],
    )(e4, counts, loc4)


def _sc_worker_id():
    info = plsc.get_sparse_core_info()
    return (lax.axis_index("s") * info.num_cores + lax.axis_index("c"),
            info.num_cores * info.num_subcores)


_SCAT_CHUNK = 128
_GATH_CHUNK = 64


def _scatter_rows(x, pos, out_rows):
    n, dim = x.shape
    nchunk_total = n // _SCAT_CHUNK
    pos2d = pos.reshape(nchunk_total, _SCAT_CHUNK)
    mesh = plsc.VectorSubcoreMesh(core_axis_name="c", subcore_axis_name="s")

    @functools.partial(
        pl.kernel, mesh=mesh,
        out_type=jax.ShapeDtypeStruct((out_rows, dim), x.dtype),
        scratch_types=[
            pltpu.VMEM((nchunk_total // 32, _SCAT_CHUNK), jnp.int32),
            pltpu.VMEM((_SCAT_CHUNK, dim), x.dtype),
            pltpu.VMEM((_SCAT_CHUNK, dim), x.dtype),
            pltpu.SemaphoreType.DMA,
        ],
    )
    def k(x_hbm, pos_hbm, out_hbm, idx_v, rows0, rows1, sem):
        wid, nw = _sc_worker_id()
        nchunks = nchunk_total // nw
        crow = wid * nchunks
        pltpu.sync_copy(pos_hbm.at[pl.ds(crow, nchunks)], idx_v)
        bufs = (rows0, rows1)
        pltpu.sync_copy(x_hbm.at[pl.ds(crow * _SCAT_CHUNK, _SCAT_CHUNK)],
                        rows0)
        for it in range(nchunks):
            h = pltpu.async_copy(bufs[it % 2], out_hbm.at[idx_v.at[it]], sem)
            if it + 1 < nchunks:
                nxt = (crow + it + 1) * _SCAT_CHUNK
                pltpu.sync_copy(x_hbm.at[pl.ds(nxt, _SCAT_CHUNK)],
                                bufs[(it + 1) % 2])
            h.wait()

    return k(x, pos2d)


def _gather_rows(ys, pos, n):
    dim = ys.shape[1]
    nchunk_total = n // _GATH_CHUNK
    pos2d = pos.reshape(nchunk_total, _GATH_CHUNK)
    mesh = plsc.VectorSubcoreMesh(core_axis_name="c", subcore_axis_name="s")

    @functools.partial(
        pl.kernel, mesh=mesh,
        out_type=jax.ShapeDtypeStruct((n, dim), ys.dtype),
        scratch_types=[
            pltpu.VMEM((nchunk_total // 32, _GATH_CHUNK), jnp.int32),
            pltpu.VMEM((_GATH_CHUNK, dim), ys.dtype),
            pltpu.VMEM((_GATH_CHUNK, dim), ys.dtype),
            pltpu.SemaphoreType.DMA,
        ],
    )
    def k(ys_hbm, pos_hbm, out_hbm, idx_v, rows0, rows1, sem):
        wid, nw = _sc_worker_id()
        nchunks = nchunk_total // nw
        crow = wid * nchunks
        pltpu.sync_copy(pos_hbm.at[pl.ds(crow, nchunks)], idx_v)
        bufs = (rows0, rows1)
        h = pltpu.async_copy(ys_hbm.at[idx_v.at[0]], rows0, sem)
        for it in range(nchunks):
            h.wait()
            if it + 1 < nchunks:
                h = pltpu.async_copy(ys_hbm.at[idx_v.at[it + 1]],
                                     bufs[(it + 1) % 2], sem)
            out_base = (crow + it) * _GATH_CHUNK
            pltpu.sync_copy(bufs[it % 2],
                            out_hbm.at[pl.ds(out_base, _GATH_CHUNK)])

    return k(ys, pos2d)


def _mm_kernel(g_ref, xs_ref, wc_ref, bc_ref, ys_ref):
    v = lax.bitcast_convert_type(xs_ref[...], jnp.uint32)
    xlo = lax.bitcast_convert_type(v << 16, jnp.float32).astype(jnp.bfloat16)
    xhi = lax.bitcast_convert_type(v & jnp.uint32(0xFFFF0000),
                                   jnp.float32).astype(jnp.bfloat16)
    acc = jnp.dot(xlo, wc_ref[0, 0], preferred_element_type=jnp.float32)
    acc += jnp.dot(xhi, wc_ref[0, 1], preferred_element_type=jnp.float32)
    ys_ref[...] = acc + bc_ref[0]


def _grouped_matmul(xs, g, W_comb, b_comb):
    grid_spec = pltpu.PrefetchScalarGridSpec(
        num_scalar_prefetch=1,
        grid=(NBLK,),
        in_specs=[
            pl.BlockSpec((BT, HALF_K), lambda b, g: (b, 0)),
            pl.BlockSpec((1, 2, HALF_K, HIDDEN_DIM),
                         lambda b, g: (g[b], 0, 0, 0)),
            pl.BlockSpec((1, 1, HIDDEN_DIM), lambda b, g: (g[b], 0, 0)),
        ],
        out_specs=pl.BlockSpec((BT, HIDDEN_DIM), lambda b, g: (b, 0)),
    )
    return pl.pallas_call(
        _mm_kernel,
        grid_spec=grid_spec,
        out_shape=jax.ShapeDtypeStruct((PAD_N, HIDDEN_DIM), jnp.float32),
    )(g, xs, W_comb, b_comb)


def kernel(x, W_shared, b_shared, W_experts, b_experts, W_router, b_router):
    W_comb, b_comb = _combine_weights(W_shared, b_shared, W_experts,
                                      b_experts)
    e, counts, local, xb = _route(x, W_router, b_router)
    pos3, g2 = _positions(e, counts, local)
    pos = pos3.reshape(NUM_TOKENS)
    g = g2.reshape(NBLK)
    xs = _scatter_rows(xb, pos, PAD_N)
    ys = _grouped_matmul(xs, g, W_comb, b_comb)
    out = _gather_rows(ys, pos, NUM_TOKENS)
    return out
